# Initial kernel scaffold; baseline (speedup 1.0000x reference)
import jax
import jax.numpy as jnp
from jax.experimental import pallas as pl


def kernel(x, weight):
    raise NotImplementedError("write your pallas kernel here")


# speedup vs baseline: 1.3064x; 1.3064x over previous
import functools

import jax
import jax.numpy as jnp
from jax import lax
from jax.experimental import pallas as pl
from jax.experimental.pallas import tpu as pltpu
from jax.experimental.pallas import tpu_sc as plsc

NUM_CORES = 2
NUM_SUBCORES = 16
NUM_WORKERS = NUM_CORES * NUM_SUBCORES
CHUNK = 128
EMB = 32


@functools.partial(jax.jit, static_argnums=())
def _sc_gather(weight, idx2d):
    n_chunks = idx2d.shape[0]
    chunks_per_w = n_chunks // NUM_WORKERS
    rows_per_w = chunks_per_w * CHUNK

    mesh = plsc.VectorSubcoreMesh(core_axis_name="c", subcore_axis_name="s")

    @functools.partial(
        pl.kernel,
        mesh=mesh,
        out_type=jax.ShapeDtypeStruct((n_chunks * CHUNK, EMB), jnp.float32),
        scratch_types=[
            pltpu.VMEM((chunks_per_w, CHUNK), jnp.int32),
            pltpu.VMEM((CHUNK, EMB), jnp.float32),
            pltpu.SemaphoreType.DMA,
        ],
        compiler_params=pltpu.CompilerParams(use_tc_tiling_on_sc=False),
    )
    def k(table_hbm, idx_hbm, out_hbm, idx_v, rows_v, sem):
        wid = lax.axis_index("s") * NUM_CORES + lax.axis_index("c")
        chunk_base = wid * chunks_per_w
        row_base = wid * rows_per_w

        pltpu.sync_copy(idx_hbm.at[pl.ds(chunk_base, chunks_per_w)], idx_v)

        def body(j, carry):
            pltpu.async_copy(table_hbm.at[idx_v.at[j]], rows_v, sem).wait()
            pltpu.sync_copy(
                rows_v, out_hbm.at[pl.ds(row_base + j * CHUNK, CHUNK)]
            )
            return carry

        lax.fori_loop(0, chunks_per_w, body, 0)

    return k(weight, idx2d)


def kernel(x, weight):
    b, s = x.shape
    idx2d = x.reshape(-1).astype(jnp.int32).reshape(-1, CHUNK)
    out = _sc_gather(weight, idx2d)
    return out.reshape(b, s, EMB)


# speedup vs baseline: 1.4932x; 1.1430x over previous
import functools

import jax
import jax.numpy as jnp
from jax import lax
from jax.experimental import pallas as pl
from jax.experimental.pallas import tpu as pltpu
from jax.experimental.pallas import tpu_sc as plsc

NUM_CORES = 2
NUM_SUBCORES = 16
NUM_WORKERS = NUM_CORES * NUM_SUBCORES
CHUNK = 128
GCH = 8
SUP = CHUNK * GCH
EMB = 32


@jax.jit
def _sc_gather(weight, idx2d):
    n_chunks = idx2d.shape[0]
    chunks_per_w = n_chunks // NUM_WORKERS
    nsup = chunks_per_w // GCH
    rows_per_w = chunks_per_w * CHUNK

    mesh = plsc.VectorSubcoreMesh(core_axis_name="c", subcore_axis_name="s")

    @functools.partial(
        pl.kernel,
        mesh=mesh,
        out_type=jax.ShapeDtypeStruct((n_chunks * CHUNK, EMB), jnp.float32),
        scratch_types=[
            pltpu.VMEM((chunks_per_w, CHUNK), jnp.int32),
            pltpu.VMEM((2, SUP, EMB), jnp.float32),
            pltpu.SemaphoreType.DMA((2,)),
            pltpu.SemaphoreType.DMA((2,)),
        ],
        compiler_params=pltpu.CompilerParams(use_tc_tiling_on_sc=False),
    )
    def k(table_hbm, idx_hbm, out_hbm, idx_v, rows_v, gsem, psem):
        wid = lax.axis_index("s") * NUM_CORES + lax.axis_index("c")
        chunk_base = wid * chunks_per_w
        row_base = wid * rows_per_w

        pltpu.sync_copy(idx_hbm.at[pl.ds(chunk_base, chunks_per_w)], idx_v)

        def wait_put(b):
            pltpu.make_async_copy(
                rows_v.at[b], out_hbm.at[pl.ds(0, SUP)], psem.at[b]
            ).wait()

        def do_super(s, b):
            handles = [
                pltpu.async_copy(
                    table_hbm.at[idx_v.at[s * GCH + g]],
                    rows_v.at[b, pl.ds(g * CHUNK, CHUNK)],
                    gsem.at[b],
                )
                for g in range(GCH)
            ]
            for h in handles:
                h.wait()
            pltpu.async_copy(
                rows_v.at[b], out_hbm.at[pl.ds(row_base + s * SUP, SUP)],
                psem.at[b],
            )

        do_super(0, 0)
        do_super(1, 1)

        def body(s, carry):
            b = lax.rem(s, 2)
            wait_put(b)
            do_super(s, b)
            return carry

        lax.fori_loop(2, nsup, body, 0)

        wait_put(0)
        wait_put(1)

    return k(weight, idx2d)


def kernel(x, weight):
    b, s = x.shape
    idx2d = x.reshape(-1).astype(jnp.int32).reshape(-1, CHUNK)
    out = _sc_gather(weight, idx2d)
    return out.reshape(b, s, EMB)


# speedup vs baseline: 1.5011x; 1.0053x over previous
import functools

import jax
import jax.numpy as jnp
from jax import lax
from jax.experimental import pallas as pl
from jax.experimental.pallas import tpu as pltpu
from jax.experimental.pallas import tpu_sc as plsc

NUM_CORES = 2
NUM_SUBCORES = 16
NUM_WORKERS = NUM_CORES * NUM_SUBCORES
CHUNK = 128
GCH = 8
SUP = CHUNK * GCH
EMB = 32


@jax.jit
def _sc_gather(weight, idx2d):
    n_chunks = idx2d.shape[0]
    chunks_per_w = n_chunks // NUM_WORKERS
    nsup = chunks_per_w // GCH
    rows_per_w = chunks_per_w * CHUNK

    mesh = plsc.VectorSubcoreMesh(core_axis_name="c", subcore_axis_name="s")

    @functools.partial(
        pl.kernel,
        mesh=mesh,
        out_type=jax.ShapeDtypeStruct((n_chunks * CHUNK, EMB), jnp.float32),
        scratch_types=[
            pltpu.VMEM((chunks_per_w, CHUNK), jnp.int32),
            pltpu.VMEM((2, SUP, EMB), jnp.float32),
            pltpu.SemaphoreType.DMA((2,)),
            pltpu.SemaphoreType.DMA((2,)),
        ],
        compiler_params=pltpu.CompilerParams(use_tc_tiling_on_sc=False),
    )
    def k(table_hbm, idx_hbm, out_hbm, idx_v, rows_v, gsem, psem):
        wid = lax.axis_index("s") * NUM_CORES + lax.axis_index("c")
        chunk_base = wid * chunks_per_w
        row_base = wid * rows_per_w

        pltpu.sync_copy(idx_hbm.at[pl.ds(chunk_base, chunks_per_w)], idx_v)

        def fire_gathers(s, b):
            for g in range(GCH):
                pltpu.async_copy(
                    table_hbm.at[idx_v.at[s * GCH + g]],
                    rows_v.at[b, pl.ds(g * CHUNK, CHUNK)],
                    gsem.at[b],
                )

        def wait_gathers(b):
            pltpu.make_async_copy(
                out_hbm.at[pl.ds(0, SUP)], rows_v.at[b], gsem.at[b]
            ).wait()

        def fire_put(s, b):
            pltpu.async_copy(
                rows_v.at[b], out_hbm.at[pl.ds(row_base + s * SUP, SUP)],
                psem.at[b],
            )

        def wait_put(b):
            pltpu.make_async_copy(
                rows_v.at[b], out_hbm.at[pl.ds(0, SUP)], psem.at[b]
            ).wait()

        fire_gathers(0, 0)
        fire_gathers(1, 1)
        wait_gathers(0)
        fire_put(0, 0)

        def body(s, carry):
            b = lax.rem(s, 2)
            b2 = 1 - b
            wait_put(b)
            fire_gathers(s, b)
            wait_gathers(b2)
            fire_put(s - 1, b2)
            return carry

        lax.fori_loop(2, nsup, body, 0)

        b_last = (nsup - 1) % 2
        wait_gathers(b_last)
        fire_put(nsup - 1, b_last)
        wait_put(1 - b_last)
        wait_put(b_last)

    return k(weight, idx2d)


def kernel(x, weight):
    b, s = x.shape
    idx2d = x.reshape(-1).astype(jnp.int32).reshape(-1, CHUNK)
    out = _sc_gather(weight, idx2d)
    return out.reshape(b, s, EMB)
